# baseline (speedup 1.0000x reference)
import functools

import jax
import jax.numpy as jnp
from jax import lax
from jax.experimental import pallas as pl
from jax.experimental.pallas import tpu as pltpu

N = 32768
B = 16
GRID = 256.0
D_OUT = 128
ROWS_2D = N // 128

BLK = 8192
SUB = BLK // 128
NBLK = N // BLK


def _body(off_ref, p0, p1, p2, p3, w_ref, out_ref, m0s, m1s, xn_s, yn_s, bi_s):
    i = pl.program_id(0)

    @pl.when(i == 0)
    def _seg_maxes():
        x = (p0[...] + p2[...]) * 0.5
        y = (p1[...] + p3[...]) * 0.5
        ii = lax.broadcasted_iota(jnp.int32, (ROWS_2D, 128), 0)
        jj = lax.broadcasted_iota(jnp.int32, (ROWS_2D, 128), 1)
        r = ii * 128 + jj
        seg = jnp.zeros((ROWS_2D, 128), jnp.int32)
        for k in range(B):
            seg = seg + (r >= off_ref[k]).astype(jnp.int32)
        neg = jnp.float32(-1e30)
        for k in range(B):
            mask = seg == k
            m0s[k] = jnp.max(jnp.where(mask, x, neg))
            m1s[k] = jnp.max(jnp.where(mask, y, neg))
        m0v = jnp.zeros((ROWS_2D, 128), jnp.float32)
        m1v = jnp.zeros((ROWS_2D, 128), jnp.float32)
        for k in range(B):
            mask = seg == k
            m0v = jnp.where(mask, m0s[k], m0v)
            m1v = jnp.where(mask, m1s[k], m1v)
        xn_s[...] = x / m0v * GRID
        yn_s[...] = y / m1v * GRID
        bi_s[...] = seg.astype(jnp.float32)

    a0 = p0[pl.ds(i * SUB, SUB), :]
    a1 = p1[pl.ds(i * SUB, SUB), :]
    a2 = p2[pl.ds(i * SUB, SUB), :]
    a3 = p3[pl.ds(i * SUB, SUB), :]
    wdt = a2 - a0
    hgt = a3 - a1
    area = wdt * hgt
    xn = xn_s[pl.ds(i * SUB, SUB), :]
    yn = yn_s[pl.ds(i * SUB, SUB), :]
    bi = bi_s[pl.ds(i * SUB, SUB), :]
    ones = jnp.ones((SUB, 128), jnp.float32)
    feats = [bi, xn, yn, a0, a1, a2, a3, wdt, hgt, area, ones]
    wmat = w_ref[...]
    for g in range(SUB):
        ft = jnp.concatenate([f[g : g + 1, :] for f in feats], axis=0)
        o = lax.dot_general(
            ft, wmat, (((0,), (0,)), ((), ())),
            preferred_element_type=jnp.float32,
        )
        out_ref[pl.ds(g * 128, 128), :] = jnp.maximum(o, 0.0)


@jax.jit
def kernel(pc0, pc1, pc2, pc3, offset, W, b):
    pcs2d = [a.reshape(ROWS_2D, 128) for a in (pc0, pc1, pc2, pc3)]
    w2 = jnp.concatenate([W, b.reshape(1, D_OUT)], axis=0)
    out = pl.pallas_call(
        _body,
        grid_spec=pltpu.PrefetchScalarGridSpec(
            num_scalar_prefetch=1,
            grid=(NBLK,),
            in_specs=[pl.BlockSpec((ROWS_2D, 128), lambda i, *_: (0, 0))] * 4
            + [pl.BlockSpec((11, D_OUT), lambda i, *_: (0, 0))],
            out_specs=pl.BlockSpec((BLK, D_OUT), lambda i, *_: (i, 0)),
            scratch_shapes=[
                pltpu.SMEM((B,), jnp.float32),
                pltpu.SMEM((B,), jnp.float32),
                pltpu.VMEM((ROWS_2D, 128), jnp.float32),
                pltpu.VMEM((ROWS_2D, 128), jnp.float32),
                pltpu.VMEM((ROWS_2D, 128), jnp.float32),
            ],
        ),
        out_shape=jax.ShapeDtypeStruct((N, D_OUT), jnp.float32),
    )(offset, *pcs2d, w2)
    return out


# speedup vs baseline: 1.1235x; 1.1235x over previous
import functools

import jax
import jax.numpy as jnp
from jax import lax
from jax.experimental import pallas as pl
from jax.experimental.pallas import tpu as pltpu

N = 32768
B = 16
GRID = 256.0
D_OUT = 128
ROWS_2D = N // 128

BLK = 8192
SUB = BLK // 128
NBLK = N // BLK


def _body(off_ref, p0, p1, p2, p3, w_ref, b_ref, out_ref, m0s, m1s, xn_s, yn_s, bi_s):
    i = pl.program_id(0)

    @pl.when(i == 0)
    def _seg_maxes():
        x = (p0[...] + p2[...]) * 0.5
        y = (p1[...] + p3[...]) * 0.5
        ii = lax.broadcasted_iota(jnp.int32, (ROWS_2D, 128), 0)
        jj = lax.broadcasted_iota(jnp.int32, (ROWS_2D, 128), 1)
        r = ii * 128 + jj
        seg = jnp.zeros((ROWS_2D, 128), jnp.int32)
        for k in range(B):
            seg = seg + (r >= off_ref[k]).astype(jnp.int32)
        neg = jnp.float32(-1e30)
        for k in range(B):
            mask = seg == k
            m0s[k] = jnp.max(jnp.where(mask, x, neg))
            m1s[k] = jnp.max(jnp.where(mask, y, neg))
        m0v = jnp.zeros((ROWS_2D, 128), jnp.float32)
        m1v = jnp.zeros((ROWS_2D, 128), jnp.float32)
        for k in range(B):
            mask = seg == k
            m0v = jnp.where(mask, m0s[k], m0v)
            m1v = jnp.where(mask, m1s[k], m1v)
        xn_s[...] = x / m0v * GRID
        yn_s[...] = y / m1v * GRID
        bi_s[...] = seg.astype(jnp.float32)

    a0 = p0[pl.ds(i * SUB, SUB), :]
    a1 = p1[pl.ds(i * SUB, SUB), :]
    a2 = p2[pl.ds(i * SUB, SUB), :]
    a3 = p3[pl.ds(i * SUB, SUB), :]
    wdt = a2 - a0
    hgt = a3 - a1
    area = wdt * hgt
    xn = xn_s[pl.ds(i * SUB, SUB), :]
    yn = yn_s[pl.ds(i * SUB, SUB), :]
    bi = bi_s[pl.ds(i * SUB, SUB), :]
    feats = [bi, xn, yn, a0, a1, a2, a3, wdt, hgt, area]
    wmat = w_ref[...]
    bvec = b_ref[...]
    for g in range(SUB):
        ft = jnp.concatenate([f[g : g + 1, :] for f in feats], axis=0)
        o = lax.dot_general(
            ft, wmat, (((0,), (0,)), ((), ())),
            preferred_element_type=jnp.float32,
        )
        out_ref[pl.ds(g * 128, 128), :] = jnp.maximum(o + bvec, 0.0)


@jax.jit
def kernel(pc0, pc1, pc2, pc3, offset, W, b):
    pcs2d = [a.reshape(ROWS_2D, 128) for a in (pc0, pc1, pc2, pc3)]
    out = pl.pallas_call(
        _body,
        grid_spec=pltpu.PrefetchScalarGridSpec(
            num_scalar_prefetch=1,
            grid=(NBLK,),
            in_specs=[pl.BlockSpec((ROWS_2D, 128), lambda i, *_: (0, 0))] * 4
            + [
                pl.BlockSpec((10, D_OUT), lambda i, *_: (0, 0)),
                pl.BlockSpec((1, D_OUT), lambda i, *_: (0, 0)),
            ],
            out_specs=pl.BlockSpec((BLK, D_OUT), lambda i, *_: (i, 0)),
            scratch_shapes=[
                pltpu.SMEM((B,), jnp.float32),
                pltpu.SMEM((B,), jnp.float32),
                pltpu.VMEM((ROWS_2D, 128), jnp.float32),
                pltpu.VMEM((ROWS_2D, 128), jnp.float32),
                pltpu.VMEM((ROWS_2D, 128), jnp.float32),
            ],
        ),
        out_shape=jax.ShapeDtypeStruct((N, D_OUT), jnp.float32),
    )(offset, *pcs2d, W, b.reshape(1, D_OUT))
    return out
